# baseline (speedup 1.0000x reference)
import functools

import jax
import jax.numpy as jnp
from jax import lax
from jax.experimental import pallas as pl
from jax.experimental.pallas import tpu as pltpu
from jax.experimental.pallas import tpu_sc as plsc

BATCH = 4096
SEQ = 200
EMBED = 64
VOCAB = 1000000
LANES = 16
NUM_CORES = 2
NUM_SUBCORES = 16
NUM_WORKERS = NUM_CORES * NUM_SUBCORES
ROWS = BATCH * SEQ
ROWS_PER_WORKER = ROWS // NUM_WORKERS
CHUNK = 128
NCHUNKS = ROWS_PER_WORKER // CHUNK
PCH = 32768

_mesh = plsc.VectorSubcoreMesh(
    core_axis_name="c", subcore_axis_name="s",
    num_cores=NUM_CORES, num_subcores=NUM_SUBCORES,
)


@functools.partial(
    pl.kernel,
    out_type=jax.ShapeDtypeStruct((ROWS, EMBED), jnp.float32),
    mesh=_mesh,
    scratch_types=[
        pltpu.VMEM((SEQ, EMBED), jnp.float32),
        pltpu.VMEM((104, CHUNK), jnp.int32),
        pltpu.VMEM((3, CHUNK, 2 * EMBED), jnp.float32),
        pltpu.VMEM((2, CHUNK, EMBED), jnp.float32),
        pltpu.SemaphoreType.DMA((3,)),
        pltpu.SemaphoreType.DMA((2,)),
    ],
    compiler_params=pltpu.CompilerParams(needs_layout_passes=False),
)
def _embed_sc(idx2_hbm, table_hbm, pos_hbm, out_hbm,
              pos_v, idx_v, rows_v, ow_v, gsem, osem):
    wid = lax.axis_index("s") * NUM_CORES + lax.axis_index("c")
    base = wid * ROWS_PER_WORKER
    pltpu.sync_copy(pos_hbm, pos_v)
    pltpu.sync_copy(idx2_hbm.at[pl.ds(wid * NCHUNKS, 104)], idx_v)
    pltpu.async_copy(
        table_hbm.at[idx_v.at[0]], rows_v.at[0], gsem.at[0])

    half = 104

    def chunk_body(i, carry):
        p = lax.rem(i, 3)
        q = lax.rem(i + 1, 3)
        w = lax.rem(i, 2)
        ii = lax.select(i < half, i, i - half)

        @pl.when(jnp.logical_and(i + 1 < NCHUNKS, i != half - 1))
        def _prefetch():
            nxt = lax.select(i + 1 < half, i + 1, i + 1 - half)
            pltpu.async_copy(
                table_hbm.at[idx_v.at[nxt]], rows_v.at[q], gsem.at[q])

        pltpu.make_async_copy(
            table_hbm.at[idx_v.at[ii]], rows_v.at[p], gsem.at[p]).wait()

        @pl.when(i == half - 1)
        def _restage():
            pltpu.sync_copy(
                idx2_hbm.at[pl.ds(wid * NCHUNKS + half, NCHUNKS - half)],
                idx_v.at[pl.ds(0, NCHUNKS - half)])
            pltpu.async_copy(
                table_hbm.at[idx_v.at[0]], rows_v.at[q], gsem.at[q])

        @pl.when(i >= 2)
        def _free():
            pltpu.make_async_copy(
                ow_v.at[w],
                out_hbm.at[pl.ds(base + (i - 2) * CHUNK, CHUNK)],
                osem.at[w]).wait()

        off = lax.rem(i * CHUNK, SEQ)
        n1 = lax.min(SEQ - off, CHUNK)

        def add_row(r, srow):
            for k in range(EMBED // LANES):
                sl = pl.ds(k * LANES, LANES)
                ow_v[w, r, sl] = rows_v[p, r, sl] + pos_v[srow, sl]

        @plsc.parallel_loop(0, n1, 1, unroll=2)
        def _seg1(r):
            add_row(r, off + r)

        @plsc.parallel_loop(n1, CHUNK, 1, unroll=2)
        def _seg2(r):
            add_row(r, off + r - SEQ)

        pltpu.async_copy(
            ow_v.at[w], out_hbm.at[pl.ds(base + i * CHUNK, CHUNK)],
            osem.at[w])
        return carry

    lax.fori_loop(0, NCHUNKS, chunk_body, 0)
    for j in (NCHUNKS - 2, NCHUNKS - 1):
        pltpu.make_async_copy(
            ow_v.at[j % 2],
            out_hbm.at[pl.ds(base + j * CHUNK, CHUNK)],
            osem.at[j % 2]).wait()


def _pack_tc_body(t_ref, o_ref):
    x = t_ref[...]
    o_ref[...] = jnp.concatenate(
        [jnp.transpose(x), jnp.zeros((PCH, EMBED), jnp.float32)], axis=1)


def kernel(input, word_table, pos_table):
    table128 = pl.pallas_call(
        _pack_tc_body,
        grid=(-(-VOCAB // PCH),),
        in_specs=[pl.BlockSpec((EMBED, PCH), lambda i: (0, i))],
        out_specs=pl.BlockSpec((PCH, 2 * EMBED), lambda i: (i, 0)),
        out_shape=jax.ShapeDtypeStruct((VOCAB, 2 * EMBED), jnp.float32),
    )(word_table.T)
    idx2 = input.reshape(-1).astype(jnp.int32).reshape(ROWS // CHUNK, CHUNK)
    flat = _embed_sc(idx2, table128, pos_table)
    return flat.reshape(BATCH, SEQ, EMBED)
